# Initial kernel scaffold; baseline (speedup 1.0000x reference)
import jax
import jax.numpy as jnp
from jax.experimental import pallas as pl


def kernel(h, m, rbf3, cbf3_0, cbf3_1, rbf_h, params, Kidx3, id_swap, id3_expand_ba, id3_reduce_ca, id_c, id_a):
    raise NotImplementedError("write your pallas kernel here")


# speedup vs baseline: 1.3322x; 1.3322x over previous
import functools

import jax
import jax.numpy as jnp
from jax import lax
from jax.experimental import pallas as pl
from jax.experimental.pallas import tpu as pltpu
from jax.experimental.pallas import tpu_sc as plsc

N_ATOMS = 10000
N_EDGES = 160000
N_TRIP = 640000
KMAX = 8
NSPH = 7
EA = 128
EE = 256
ET = 64
ERBF = 16
ECBF = 16
EBIL = 64
INV_SQRT_2 = 1.0 / (2.0 ** 0.5)
F32 = jnp.float32
I32 = jnp.int32

NC = 2
NS = 16
NW = NC * NS
EPW = N_EDGES // NW
SPW = EPW * KMAX
NZROW = 64
TCHUNK = 1024
TRIP_PAD = N_TRIP + TCHUNK + 8


def _act(y):
    return y * jax.nn.sigmoid(y) * (1.0 / 0.6)


def _dot(a, b):
    return jnp.dot(a, b, preferred_element_type=F32)


BE_A = 1600
BE_C = 128
BE_E = 1280
BE_I = 1280
BA_G = 400


def _tca_body(m_ref, rbf3_ref, wdba_ref, wmrbf_ref, wdown_ref, xdown_ref):
    t = _act(_dot(m_ref[...], wdba_ref[...]))
    t = t * _dot(rbf3_ref[...], wmrbf_ref[...])
    xdown_ref[...] = _act(_dot(t, wdown_ref[...]))


def _tc_a(m, rbf3, wdba, wmrbf, wdown):
    g = N_EDGES // BE_A
    return pl.pallas_call(
        _tca_body,
        grid=(g,),
        in_specs=[
            pl.BlockSpec((BE_A, EE), lambda i: (i, 0)),
            pl.BlockSpec((BE_A, ERBF), lambda i: (i, 0)),
            pl.BlockSpec((EE, EE), lambda i: (0, 0)),
            pl.BlockSpec((ERBF, EE), lambda i: (0, 0)),
            pl.BlockSpec((EE, ET), lambda i: (0, 0)),
        ],
        out_specs=pl.BlockSpec((BE_A, ET), lambda i: (i, 0)),
        out_shape=jax.ShapeDtypeStruct((N_EDGES, ET), F32),
    )(m, rbf3, wdba, wmrbf, wdown)


def _tcc_body(m2_ref, cbf1_ref, cbf0_ref, wbil_ref, xt_ref):
    m2 = m2_ref[...]
    w1 = cbf1_ref[...]
    w0 = cbf0_ref[...]
    sumk = []
    for s in range(NSPH):
        acc = w1[:, s, 0][:, None] * m2[:, 0, :]
        for k in range(1, KMAX):
            acc = acc + w1[:, s, k][:, None] * m2[:, k, :]
        sumk.append(acc)
    xt = None
    for c in range(ECBF):
        rw = w0[:, c, 0][:, None] * sumk[0]
        for s in range(1, NSPH):
            rw = rw + w0[:, c, s][:, None] * sumk[s]
        part = _dot(rw, wbil_ref[c])
        xt = part if xt is None else xt + part
    xt_ref[...] = xt


def _tc_c(m2, cbf3_1, cbf3_0, wbil_t):
    g = N_EDGES // BE_C
    return pl.pallas_call(
        _tcc_body,
        grid=(g,),
        in_specs=[
            pl.BlockSpec((BE_C, KMAX, ET), lambda i: (i, 0, 0)),
            pl.BlockSpec((BE_C, NSPH, KMAX), lambda i: (i, 0, 0)),
            pl.BlockSpec((BE_C, ECBF, NSPH), lambda i: (i, 0, 0)),
            pl.BlockSpec((ECBF, ET, EBIL), lambda i: (0, 0, 0)),
        ],
        out_specs=pl.BlockSpec((BE_C, EBIL), lambda i: (i, 0)),
        out_shape=jax.ShapeDtypeStruct((N_EDGES, EBIL), F32),
    )(m2, cbf3_1, cbf3_0, wbil_t)


def _res_chain(x, pairs):
    for (w1, w2) in pairs:
        y = _act(_dot(_act(_dot(x, w1)), w2))
        x = (x + y) * INV_SQRT_2
    return x


def _tce_body(m_ref, xt_ref, xtsw_ref, rbfh_ref, wupca_ref, wupac_ref,
              wdca_ref, wbs1_ref, wbs2_ref, was11_ref, was12_ref,
              was21_ref, was22_ref, warbf_ref, mnew_ref, xapre_ref):
    m = m_ref[...]
    x_ca = _act(_dot(xt_ref[...], wupca_ref[...]))
    x_ac = _act(_dot(xtsw_ref[...], wupac_ref[...]))
    x3 = (x_ca + x_ac) * INV_SQRT_2
    x = (_act(_dot(m, wdca_ref[...])) + x3) * INV_SQRT_2
    x = _res_chain(x, [(wbs1_ref[...], wbs2_ref[...])])
    mnew = (m + x) * INV_SQRT_2
    mnew = _res_chain(mnew, [(was11_ref[...], was12_ref[...]),
                             (was21_ref[...], was22_ref[...])])
    mnew_ref[...] = mnew
    xapre_ref[...] = mnew * _dot(rbfh_ref[...], warbf_ref[...])


def _tc_e(m, xt, xtsw, rbfh, wupca, wupac, wdca, wbs, was, warbf):
    g = N_EDGES // BE_E
    ew = lambda i: (i, 0)
    w0 = lambda i: (0, 0)
    return pl.pallas_call(
        _tce_body,
        grid=(g,),
        in_specs=[
            pl.BlockSpec((BE_E, EE), ew),
            pl.BlockSpec((BE_E, EBIL), ew),
            pl.BlockSpec((BE_E, EBIL), ew),
            pl.BlockSpec((BE_E, ERBF), ew),
            pl.BlockSpec((EBIL, EE), w0),
            pl.BlockSpec((EBIL, EE), w0),
            pl.BlockSpec((EE, EE), w0),
            pl.BlockSpec((EE, EE), w0),
            pl.BlockSpec((EE, EE), w0),
            pl.BlockSpec((EE, EE), w0),
            pl.BlockSpec((EE, EE), w0),
            pl.BlockSpec((EE, EE), w0),
            pl.BlockSpec((EE, EE), w0),
            pl.BlockSpec((ERBF, EE), w0),
        ],
        out_specs=[pl.BlockSpec((BE_E, EE), ew), pl.BlockSpec((BE_E, EE), ew)],
        out_shape=[jax.ShapeDtypeStruct((N_EDGES, EE), F32),
                   jax.ShapeDtypeStruct((N_EDGES, EE), F32)],
    )(m, xt, xtsw, rbfh, wupca, wupac, wdca, wbs[0][0], wbs[0][1],
      was[0][0], was[0][1], was[1][0], was[1][1], warbf)


def _tcg_body(h_ref, xa_ref, wa1_ref, wr1_ref, wr2_ref, wr3_ref, wr4_ref,
              wr5_ref, wr6_ref, hnew_ref):
    xa = _act(_dot(xa_ref[...], wa1_ref[...]))
    xa = _res_chain(xa, [(wr1_ref[...], wr2_ref[...]),
                         (wr3_ref[...], wr4_ref[...]),
                         (wr5_ref[...], wr6_ref[...])])
    hnew_ref[...] = (h_ref[...] + xa) * INV_SQRT_2


def _tc_g(h, xa, wa1, wares):
    g = N_ATOMS // BA_G
    aw = lambda i: (i, 0)
    w0 = lambda i: (0, 0)
    return pl.pallas_call(
        _tcg_body,
        grid=(g,),
        in_specs=[
            pl.BlockSpec((BA_G, EA), aw),
            pl.BlockSpec((BA_G, EE), aw),
            pl.BlockSpec((EE, EA), w0),
        ] + [pl.BlockSpec((EA, EA), w0)] * 6,
        out_specs=pl.BlockSpec((BA_G, EA), aw),
        out_shape=jax.ShapeDtypeStruct((N_ATOMS, EA), F32),
    )(h, xa, wa1, wares[0][0], wares[0][1], wares[1][0], wares[1][1],
      wares[2][0], wares[2][1])


def _tci_body(mnew_ref, hc_ref, ha_ref, wc1_ref, wc2_ref, wc3_ref,
              wm1_ref, wm2_ref, mout_ref):
    mnew = mnew_ref[...]
    m2o = _act(_dot(hc_ref[...], wc1_ref[...]) +
               _dot(ha_ref[...], wc2_ref[...]) +
               _dot(mnew, wc3_ref[...]))
    m2o = _res_chain(m2o, [(wm1_ref[...], wm2_ref[...])])
    mout_ref[...] = (mnew + m2o) * INV_SQRT_2


def _tc_i(mnew, hc, ha, wc1, wc2, wc3, wmres):
    g = N_EDGES // BE_I
    ew = lambda i: (i, 0)
    w0 = lambda i: (0, 0)
    return pl.pallas_call(
        _tci_body,
        grid=(g,),
        in_specs=[
            pl.BlockSpec((BE_I, EE), ew),
            pl.BlockSpec((BE_I, EA), ew),
            pl.BlockSpec((BE_I, EA), ew),
            pl.BlockSpec((EA, EE), w0),
            pl.BlockSpec((EA, EE), w0),
            pl.BlockSpec((EE, EE), w0),
            pl.BlockSpec((EE, EE), w0),
            pl.BlockSpec((EE, EE), w0),
        ],
        out_specs=pl.BlockSpec((BE_I, EE), ew),
        out_shape=jax.ShapeDtypeStruct((N_EDGES, EE), F32),
    )(mnew, hc, ha, wc1, wc2, wc3, wmres[0][0], wmres[0][1])


def _sc_mesh():
    return plsc.VectorSubcoreMesh(core_axis_name="c", subcore_axis_name="s")


def _wid():
    return lax.axis_index("c") * NS + lax.axis_index("s")


def _scb_body(xd_hbm, rc_hbm, kx_hbm, ba_hbm, off_hbm, m2_hbm,
              off_v, slot_v, rc_v, kx_v, ba_v, rows_v, sem):
    w = _wid()
    pltpu.sync_copy(off_hbm, off_v)
    start = off_v[pl.ds(w, 16)][0]
    end = off_v[pl.ds(w + 1, 16)][0]
    base_e = w * EPW

    lanes = lax.iota(I32, 16)
    lane_masks = [lanes == j for j in range(16)]

    def init_body(i, _):
        for j in range(4):
            slot_v[pl.ds(i * 64 + j * 16, 16)] = N_EDGES + j * 16 + lanes
        return 0
    lax.fori_loop(0, SPW // 64, init_body, 0, unroll=False)

    astart = (start // 8) * 8
    nchunk = (end - astart + TCHUNK - 1) // TCHUNK

    def chunk_body(ci, _):
        toff = astart + ci * TCHUNK
        pltpu.sync_copy(rc_hbm.at[pl.ds(toff, TCHUNK)], rc_v)
        pltpu.sync_copy(kx_hbm.at[pl.ds(toff, TCHUNK)], kx_v)
        pltpu.sync_copy(ba_hbm.at[pl.ds(toff, TCHUNK)], ba_v)

        def group_body(g, _):
            sl = pl.ds(g * 16, 16)
            key = (rc_v[sl] - base_e) * KMAX + kx_v[sl]
            val = ba_v[sl]
            gidx = toff + g * 16 + lanes
            inr = (gidx >= start) & (gidx < end)
            for j in range(16):
                plsc.store_scatter(slot_v, [key], val,
                                   mask=lane_masks[j] & inr)
            return 0
        lax.fori_loop(0, TCHUNK // 16, group_body, 0, unroll=False)
        return 0
    lax.fori_loop(0, nchunk, chunk_body, 0, unroll=False)

    base_slot = w * SPW

    def gather_body(g, _):
        cps = [pltpu.async_copy(
                   xd_hbm.at[slot_v.at[pl.ds((g * 5 + j) * 64, 64)]],
                   rows_v.at[pl.ds(j * 64, 64)], sem)
               for j in range(5)]
        for cp in cps:
            cp.wait()
        pltpu.sync_copy(rows_v, m2_hbm.at[pl.ds(base_slot + g * 320, 320)])
        return 0
    lax.fori_loop(0, SPW // 320, gather_body, 0, unroll=False)


def _sc_b(xd_ext, rc_pad, kx_pad, ba_pad, off):
    kern = functools.partial(
        pl.kernel,
        mesh=_sc_mesh(),
        compiler_params=pltpu.CompilerParams(needs_layout_passes=False, use_tc_tiling_on_sc=False),
        out_type=jax.ShapeDtypeStruct((N_EDGES * KMAX, ET), F32),
        scratch_types=[
            pltpu.VMEM((48,), I32),
            pltpu.VMEM((SPW,), I32),
            pltpu.VMEM((TCHUNK,), I32),
            pltpu.VMEM((TCHUNK,), I32),
            pltpu.VMEM((TCHUNK,), I32),
            pltpu.VMEM((320, ET), F32),
            pltpu.SemaphoreType.DMA,
        ],
    )(_scb_body)
    return kern(xd_ext, rc_pad, kx_pad, ba_pad, off)


def _scd_body(xt_hbm, idx_hbm, out_hbm, idx_v, rows_v, sem):
    w = _wid()
    pltpu.sync_copy(idx_hbm.at[w], idx_v)

    def outer(g, _):
        cps = [pltpu.async_copy(xt_hbm.at[idx_v.at[g * 10 + j]],
                                rows_v.at[pl.ds(j * 100, 100)], sem)
               for j in range(10)]
        for cp in cps:
            cp.wait()
        pltpu.sync_copy(rows_v,
                        out_hbm.at[pl.ds(w * EPW + g * 1000, 1000)])
        return 0
    lax.fori_loop(0, 5, outer, 0, unroll=False)


def _sc_d(xt, idx3):
    kern = functools.partial(
        pl.kernel,
        mesh=_sc_mesh(),
        compiler_params=pltpu.CompilerParams(needs_layout_passes=False, use_tc_tiling_on_sc=False),
        out_type=jax.ShapeDtypeStruct((N_EDGES, EBIL), F32),
        scratch_types=[
            pltpu.VMEM((50, 100), I32),
            pltpu.VMEM((1000, EBIL), F32),
            pltpu.SemaphoreType.DMA,
        ],
    )(_scd_body)
    return kern(xt, idx3)


FCH = 400
APS = N_ATOMS // NS


def _scf_body(xap_hbm, ida_hbm, out_hbm, zbuf_v, idx_v, rows_v, acc_sh, sem):
    cid = lax.axis_index("c")
    sid = lax.axis_index("s")

    def zb(i, _):
        for j in range(4):
            zbuf_v[i, pl.ds(j * 16, 16)] = jnp.zeros((16,), F32)
        return 0
    lax.fori_loop(0, 125, zb, 0, unroll=False)

    ebase = sid * (N_EDGES // NS)
    for p in range(2):
        col0 = p * 128 + cid * 64
        for t in range(5):
            pltpu.sync_copy(zbuf_v,
                            acc_sh.at[pl.ds(sid * APS + t * 125, 125)])
        plsc.subcore_barrier()

        def chunk(i, _):
            e0 = ebase + i * FCH
            pltpu.sync_copy(
                xap_hbm.at[pl.ds(e0, FCH), pl.ds(col0, 64)], rows_v)
            pltpu.sync_copy(ida_hbm.at[sid * 25 + i], idx_v)
            cps = [pltpu.async_copy(rows_v.at[pl.ds(j * 100, 100)],
                                    acc_sh.at[idx_v.at[j]], sem, add=True)
                   for j in range(4)]
            for cp in cps:
                cp.wait()
            return 0
        lax.fori_loop(0, (N_EDGES // NS) // FCH, chunk, 0, unroll=False)

        plsc.subcore_barrier()
        pltpu.sync_copy(acc_sh.at[pl.ds(sid * APS, APS)],
                        out_hbm.at[pl.ds(sid * APS, APS), pl.ds(col0, 64)])


def _sc_f(xa_pre, ida4):
    kern = functools.partial(
        pl.kernel,
        mesh=_sc_mesh(),
        compiler_params=pltpu.CompilerParams(needs_layout_passes=False, use_tc_tiling_on_sc=False),
        out_type=jax.ShapeDtypeStruct((N_ATOMS, EE), F32),
        scratch_types=[
            pltpu.VMEM((125, 64), F32),
            pltpu.VMEM((4, 100), I32),
            pltpu.VMEM((FCH, 64), F32),
            pltpu.VMEM_SHARED((N_ATOMS, 64), F32),
            pltpu.SemaphoreType.DMA,
        ],
    )(_scf_body)
    return kern(xa_pre, ida4)


def _sch_body(h_hbm, idc_hbm, ida_hbm, hc_hbm, ha_hbm,
              idx_v, rows_v, sem):
    w = _wid()
    for (src_idx, dst) in ((idc_hbm, hc_hbm), (ida_hbm, ha_hbm)):
        pltpu.sync_copy(src_idx.at[w], idx_v)

        def outer(g, _):
            cps = [pltpu.async_copy(h_hbm.at[idx_v.at[g * 5 + j]],
                                    rows_v.at[pl.ds(j * 100, 100)], sem)
                   for j in range(5)]
            for cp in cps:
                cp.wait()
            pltpu.sync_copy(rows_v,
                            dst.at[pl.ds(w * EPW + g * 500, 500)])
            return 0
        lax.fori_loop(0, 10, outer, 0, unroll=False)


def _sc_h(h_new, idc3, ida3):
    kern = functools.partial(
        pl.kernel,
        mesh=_sc_mesh(),
        compiler_params=pltpu.CompilerParams(needs_layout_passes=False, use_tc_tiling_on_sc=False),
        out_type=[jax.ShapeDtypeStruct((N_EDGES, EA), F32),
                  jax.ShapeDtypeStruct((N_EDGES, EA), F32)],
        scratch_types=[
            pltpu.VMEM((50, 100), I32),
            pltpu.VMEM((500, EA), F32),
            pltpu.SemaphoreType.DMA,
        ],
    )(_sch_body)
    return kern(h_new, idc3, ida3)


def kernel(h, m, rbf3, cbf3_0, cbf3_1, rbf_h, params,
           Kidx3, id_swap, id3_expand_ba, id3_reduce_ca, id_c, id_a):
    p = params

    rc = id3_reduce_ca.astype(I32)
    kx = Kidx3.astype(I32)
    ba = id3_expand_ba.astype(I32)
    bounds = (jnp.arange(NW + 1, dtype=I32) * EPW)
    off = jnp.searchsorted(rc, bounds, side="left").astype(I32)
    off = jnp.concatenate([off, jnp.zeros((48 - NW - 1,), I32)])
    rc_pad = jnp.concatenate([rc, jnp.zeros((TRIP_PAD - N_TRIP,), I32)])
    kx_pad = jnp.concatenate([kx, jnp.zeros((TRIP_PAD - N_TRIP,), I32)])
    ba_pad = jnp.concatenate([ba, jnp.zeros((TRIP_PAD - N_TRIP,), I32)])
    idsw3 = id_swap.astype(I32).reshape(NW, 50, 100)
    idc3 = id_c.astype(I32).reshape(NW, 50, 100)
    ida3 = id_a.astype(I32).reshape(NW, 50, 100)
    ida4 = id_a.astype(I32).reshape(400, 4, 100)
    wbil_t = p["W_bil"].transpose(1, 0, 2)
    wc1 = p["W_cat"][:EA]
    wc2 = p["W_cat"][EA:2 * EA]
    wc3 = p["W_cat"][2 * EA:]

    x_down = _tc_a(m, rbf3, p["W_dba"], p["W_mrbf"], p["W_down"])
    xd_ext = jnp.concatenate([x_down, jnp.zeros((NZROW, ET), F32)], axis=0)
    m2 = _sc_b(xd_ext, rc_pad, kx_pad, ba_pad, off)
    xt = _tc_c(m2.reshape(N_EDGES, KMAX, ET), cbf3_1, cbf3_0, wbil_t)
    xtsw = _sc_d(xt, idsw3)
    m_new, xa_pre = _tc_e(m, xt, xtsw, rbf_h, p["W_up_ca"], p["W_up_ac"],
                          p["W_dca"], p["W_bs"], p["W_as"], p["W_arbf"])
    xa = _sc_f(xa_pre, ida4)
    h_new = _tc_g(h, xa, p["W_a1"], p["W_ares"])
    hc, ha = _sc_h(h_new, idc3, ida3)
    m_out = _tc_i(m_new, hc, ha, wc1, wc2, wc3, p["W_mres"])
    return h_new, m_out
